# baseline (speedup 1.0000x reference)
import functools

import jax
import jax.numpy as jnp
from jax import lax
from jax.experimental import pallas as pl
from jax.experimental.pallas import tpu as pltpu
from jax.experimental.pallas import tpu_sc as plsc

N = 10000
NPAD = 10240
E = 320000
D_IN = 128
D_HID = 512
D_OUT = 2
EPS = 1e-5

NC = 2
NS = 16
NW = NC * NS
EPT = E // NW
K = 80
NCHUNK = EPT // K
U = 25
NG = NCHUNK // U
NPAIR = NG // 2
RSLOT = 4
ISLOT = 8
NBODY = (NCHUNK - 5) // ISLOT
ETAIL = NBODY * ISLOT
ZCH = NPAD // NS

def _sc_mesh():
    return plsc.VectorSubcoreMesh(
        core_axis_name="c", subcore_axis_name="s", num_cores=NC, num_subcores=NS
    )


def _deg_hist_body(ei_hbm, out_hbm, ones_v, zeros_v, idxA, idxB, deg_sp,
                   semiA, semiB, semsA, semsB):
    c = lax.axis_index("c")
    s = lax.axis_index("s")
    wid = c * NS + s

    for i in range(K // 16):
        ones_v[pl.ds(i * 16, 16)] = jnp.full((16,), 1.0, jnp.float32)
    for i in range(ZCH // 16):
        zeros_v[pl.ds(i * 16, 16)] = jnp.zeros((16,), jnp.float32)
    pltpu.sync_copy(zeros_v, deg_sp.at[pl.ds(s * ZCH, ZCH)])
    plsc.subcore_barrier()

    base = E + wid * EPT

    def iload(g, buf, b, sem):
        off = base + (g * U + b) * K
        pltpu.async_copy(ei_hbm.at[pl.ds(off, K)], buf.at[b], sem)

    def idrain(buf, sem):
        pltpu.make_async_copy(ei_hbm.at[pl.ds(0, K)], buf.at[0], sem).wait()

    def scat(buf, b, sem):
        pltpu.async_copy(ones_v, deg_sp.at[buf.at[b]], sem, add=True)

    def sdrain(buf, sem):
        pltpu.make_async_copy(ones_v, deg_sp.at[buf.at[0]], sem).wait()

    for b in range(U):
        iload(0, idxA, b, semiA)

    def body(i, carry):
        g0 = 2 * i

        @pl.when(i > 0)
        def _():
            for b in range(U):
                sdrain(idxB, semsB)

        for b in range(U):
            iload(g0 + 1, idxB, b, semiB)
        for b in range(U):
            idrain(idxA, semiA)
        for b in range(U):
            scat(idxA, b, semsA)

        for b in range(U):
            idrain(idxB, semiB)
        for b in range(U):
            scat(idxB, b, semsB)
        for b in range(U):
            sdrain(idxA, semsA)
        for b in range(U):
            iload(g0 + 2, idxA, b, semiA)
        return carry

    lax.fori_loop(0, NPAIR, body, 0)

    for b in range(U):
        idrain(idxA, semiA)
    for b in range(U):
        scat(idxA, b, semsA)
    for b in range(U):
        sdrain(idxB, semsB)
    for b in range(U):
        sdrain(idxA, semsA)

    plsc.subcore_barrier()
    pltpu.sync_copy(deg_sp.at[pl.ds(s * ZCH, ZCH)], out_hbm.at[c, pl.ds(s * ZCH, ZCH)])


@functools.cache
def _deg_hist_kernel():
    return pl.kernel(
        _deg_hist_body,
        out_type=jax.ShapeDtypeStruct((NC, NPAD), jnp.float32),
        mesh=_sc_mesh(),
        scratch_types=[
            pltpu.VMEM((K,), jnp.float32),
            pltpu.VMEM((ZCH,), jnp.float32),
            pltpu.VMEM((U, K), jnp.int32),
            pltpu.VMEM((U, K), jnp.int32),
            pltpu.VMEM_SHARED((NPAD,), jnp.float32),
            pltpu.SemaphoreType.DMA,
            pltpu.SemaphoreType.DMA,
            pltpu.SemaphoreType.DMA,
            pltpu.SemaphoreType.DMA,
        ],
    )


def _deg_hist(ei):
    return _deg_hist_kernel()(ei)


def _edge_agg_body(ei_hbm, xs_hbm, out_hbm, *refs):
    idx = refs[0:ISLOT]
    rows = refs[ISLOT:ISLOT + RSLOT]
    agg_sp = refs[ISLOT + RSLOT]
    semi = refs[ISLOT + RSLOT + 1:ISLOT + RSLOT + 1 + ISLOT]
    semg = refs[ISLOT + RSLOT + 1 + ISLOT:ISLOT + RSLOT + 1 + ISLOT + RSLOT]
    sems = refs[ISLOT + RSLOT + 1 + ISLOT + RSLOT:]

    c = lax.axis_index("c")
    s = lax.axis_index("s")
    wid = c * NS + s
    base = wid * EPT

    def zrow(i, carry):
        for j in range(D_IN // 16):
            rows[0][i, pl.ds(j * 16, 16)] = jnp.zeros((16,), jnp.float32)
        return carry

    lax.fori_loop(0, K, zrow, 0)
    for z in range(ZCH // K):
        pltpu.sync_copy(rows[0], agg_sp.at[pl.ds(s * ZCH + z * K, K)])
    plsc.subcore_barrier()

    def eload(g, sj):
        sl = idx[sj % ISLOT]
        sem = semi[sj % ISLOT]
        off = base + g * K
        pltpu.async_copy(ei_hbm.at[pl.ds(off, K)], sl.at[0], sem)
        pltpu.async_copy(ei_hbm.at[pl.ds(E + off, K)], sl.at[1], sem)

    def edrain(sj):
        sl = idx[sj % ISLOT]
        sem = semi[sj % ISLOT]
        pltpu.make_async_copy(ei_hbm.at[pl.ds(0, K)], sl.at[0], sem).wait()
        pltpu.make_async_copy(ei_hbm.at[pl.ds(0, K)], sl.at[1], sem).wait()

    def gath(sj):
        pltpu.async_copy(xs_hbm.at[idx[sj % ISLOT].at[0]], rows[sj % RSLOT],
                         semg[sj % RSLOT])

    def gdrain(sj):
        pltpu.make_async_copy(xs_hbm.at[idx[sj % ISLOT].at[0]],
                              rows[sj % RSLOT], semg[sj % RSLOT]).wait()

    def scat(sj):
        pltpu.async_copy(rows[sj % RSLOT], agg_sp.at[idx[sj % ISLOT].at[1]],
                         sems[sj % RSLOT], add=True)

    def sdrain(sj):
        pltpu.make_async_copy(rows[sj % RSLOT],
                              agg_sp.at[idx[sj % ISLOT].at[1]],
                              sems[sj % RSLOT]).wait()

    for g in range(4):
        eload(g, g)
    edrain(0)
    edrain(1)
    gath(0)
    gath(1)

    def body(i, carry):
        for jj in range(ISLOT):
            g = i * ISLOT + jj
            if jj >= 2:
                sdrain(jj - 2)
            else:

                @pl.when(i > 0)
                def _():
                    sdrain(jj - 2)

            eload(g + 4, jj + 4)
            edrain(jj + 2)
            gath(jj + 2)
            gdrain(jj)
            scat(jj)
        return carry

    lax.fori_loop(0, NBODY, body, 0)

    for g in range(ETAIL, NCHUNK):
        sdrain(g - 2)
        if g + 4 < NCHUNK:
            eload(g + 4, g + 4)
        if g + 2 < NCHUNK:
            edrain(g + 2)
            gath(g + 2)
        gdrain(g)
        scat(g)
    sdrain(NCHUNK - 2)
    sdrain(NCHUNK - 1)

    plsc.subcore_barrier()
    pltpu.sync_copy(
        agg_sp.at[pl.ds(s * ZCH, ZCH)], out_hbm.at[c, pl.ds(s * ZCH, ZCH)]
    )


@functools.cache
def _edge_agg_kernel():
    scratch = (
        [pltpu.VMEM((2, K), jnp.int32) for _ in range(ISLOT)]
        + [pltpu.VMEM((K, D_IN), jnp.float32) for _ in range(RSLOT)]
        + [pltpu.VMEM_SHARED((NPAD, D_IN), jnp.float32)]
        + [pltpu.SemaphoreType.DMA for _ in range(ISLOT + 2 * RSLOT)]
    )
    return pl.kernel(
        _edge_agg_body,
        out_type=jax.ShapeDtypeStruct((NC, NPAD, D_IN), jnp.float32),
        mesh=_sc_mesh(),
        scratch_types=scratch,
    )


def _edge_agg(ei, xs):
    return _edge_agg_kernel()(ei, xs)


_RB = 2000
_GB = N // _RB


def _prescale_body(degT_ref, x_ref, xs_ref, dinv_ref):
    deg = degT_ref[:, 0:1] + degT_ref[:, 1:2] + 1.0
    dinv = lax.rsqrt(deg)
    dinv_ref[...] = dinv
    xs_ref[...] = x_ref[...] * dinv


_RP = 5000

def _prescale(degT, x):
    return pl.pallas_call(
        _prescale_body,
        grid=(N // _RP,),
        in_specs=[
            pl.BlockSpec((_RP, NC), lambda i: (i, 0)),
            pl.BlockSpec((_RP, D_IN), lambda i: (i, 0)),
        ],
        out_specs=[
            pl.BlockSpec((_RP, D_IN), lambda i: (i, 0)),
            pl.BlockSpec((_RP, 1), lambda i: (i, 0)),
        ],
        out_shape=[
            jax.ShapeDtypeStruct((N, D_IN), jnp.float32),
            jax.ShapeDtypeStruct((N, 1), jnp.float32),
        ],
    )(degT, x)


def _dense_body(agg_ref, xs_ref, dinv_ref, w_ref, bg_ref, g_ref, be_ref,
                wl_ref, bl_ref, probs_ref, rsu_ref, ca_ref, gram_ref,
                m1_ref, rs_ref, acache_ref):
    j = pl.program_id(0)

    @pl.when(j == 0)
    def _():
        ca_ref[...] = jnp.zeros_like(ca_ref)
        gram_ref[...] = jnp.zeros_like(gram_ref)

    @pl.when(j < _GB)
    def _():
        a = (agg_ref[0] + agg_ref[1] + xs_ref[...]) * dinv_ref[...]
        acache_ref[pl.ds(j * _RB, _RB), :] = a
        ca_ref[...] += jnp.sum(a, axis=0, keepdims=True)
        gram_ref[...] += lax.dot_general(
            a, a, (((0,), (0,)), ((), ())),
            preferred_element_type=jnp.float32)

    @pl.when(j == _GB)
    def _():
        inv_n = jnp.float32(1.0 / N)
        bg = bg_ref[...]
        m1 = jnp.dot(ca_ref[...] * inv_n, w_ref[...],
                     preferred_element_type=jnp.float32) + bg
        gw = jnp.dot(gram_ref[...], w_ref[...],
                     preferred_element_type=jnp.float32)
        q = jnp.sum(w_ref[...] * gw, axis=0, keepdims=True) * inv_n
        var = q + (2.0 * m1 - bg) * bg - m1 * m1
        m1_ref[...] = m1
        rs_ref[...] = lax.rsqrt(var + EPS)

    @pl.when(j >= _GB)
    def _():
        a = acache_ref[pl.ds((j - _GB) * _RB, _RB), :]
        y = jnp.dot(a, w_ref[...], preferred_element_type=jnp.float32)
        y = y + bg_ref[...]
        z = g_ref[...] * ((y - m1_ref[...]) * rs_ref[...]) + be_ref[...]

        @pl.when(j == _GB)
        def _():
            rsu_ref[...] = z[0:1, :]

        logits = jnp.dot(z, wl_ref[...], preferred_element_type=jnp.float32)
        logits = jnp.maximum(logits + bl_ref[...], 0.0)
        m = jnp.max(logits, axis=1, keepdims=True)
        e = jnp.exp(logits - m)
        probs_ref[...] = e / jnp.sum(e, axis=1, keepdims=True)


def _dense(agg, xs, dinv, W, bg, gamma, beta, Wl, bl):
    def row(j):
        return jnp.minimum(j, _GB - 1)

    return pl.pallas_call(
        _dense_body,
        grid=(2 * _GB,),
        in_specs=[
            pl.BlockSpec((NC, _RB, D_IN), lambda j: (0, row(j), 0)),
            pl.BlockSpec((_RB, D_IN), lambda j: (row(j), 0)),
            pl.BlockSpec((_RB, 1), lambda j: (row(j), 0)),
            pl.BlockSpec((D_IN, D_HID), lambda j: (0, 0)),
            pl.BlockSpec((1, D_HID), lambda j: (0, 0)),
            pl.BlockSpec((1, D_HID), lambda j: (0, 0)),
            pl.BlockSpec((1, D_HID), lambda j: (0, 0)),
            pl.BlockSpec((D_HID, D_OUT), lambda j: (0, 0)),
            pl.BlockSpec((1, D_OUT), lambda j: (0, 0)),
        ],
        out_specs=[
            pl.BlockSpec((_RB, D_OUT), lambda j: (jnp.where(j < _GB, 0, j - _GB), 0)),
            pl.BlockSpec((1, D_HID), lambda j: (0, 0)),
        ],
        out_shape=[
            jax.ShapeDtypeStruct((N, D_OUT), jnp.float32),
            jax.ShapeDtypeStruct((1, D_HID), jnp.float32),
        ],
        scratch_shapes=[
            pltpu.VMEM((1, D_IN), jnp.float32),
            pltpu.VMEM((D_IN, D_IN), jnp.float32),
            pltpu.VMEM((1, D_HID), jnp.float32),
            pltpu.VMEM((1, D_HID), jnp.float32),
            pltpu.VMEM((N, D_IN), jnp.float32),
        ],
    )(agg, xs, dinv, W, bg, gamma, beta, Wl, bl)


def kernel(node_feature, edge_index, W_gcn, b_gcn, gamma, beta, W_lin, b_lin):
    ei = edge_index.reshape(2 * E)
    deg_p = _deg_hist(ei)
    degT = deg_p.T
    xs, dinv = _prescale(degT, node_feature)
    agg = _edge_agg(ei, xs)
    probs, rsu = _dense(agg, xs, dinv, W_gcn, b_gcn.reshape(1, D_HID),
                        gamma.reshape(1, D_HID), beta.reshape(1, D_HID),
                        W_lin, b_lin.reshape(1, D_OUT))
    return probs, rsu


# speedup vs baseline: 1.0352x; 1.0352x over previous
import functools

import jax
import jax.numpy as jnp
from jax import lax
from jax.experimental import pallas as pl
from jax.experimental.pallas import tpu as pltpu
from jax.experimental.pallas import tpu_sc as plsc

N = 10000
NPAD = 10240
E = 320000
D_IN = 128
D_HID = 512
D_OUT = 2
EPS = 1e-5

NC = 2
NS = 16
NW = NC * NS
EPT = E // NW
K = 80
NCHUNK = EPT // K
U = 25
NG = NCHUNK // U
NPAIR = NG // 2
RSLOT = 4
ISLOT = 8
NBODY = (NCHUNK - 5) // ISLOT
ETAIL = NBODY * ISLOT
ZCH = NPAD // NS

def _sc_mesh():
    return plsc.VectorSubcoreMesh(
        core_axis_name="c", subcore_axis_name="s", num_cores=NC, num_subcores=NS
    )


def _deg_hist_body(ei_hbm, out_hbm, ones_v, zeros_v, idxA, idxB, deg_sp,
                   semiA, semiB, semsA, semsB):
    c = lax.axis_index("c")
    s = lax.axis_index("s")
    wid = c * NS + s

    for i in range(K // 16):
        ones_v[pl.ds(i * 16, 16)] = jnp.full((16,), 1.0, jnp.float32)
    for i in range(ZCH // 16):
        zeros_v[pl.ds(i * 16, 16)] = jnp.zeros((16,), jnp.float32)
    pltpu.sync_copy(zeros_v, deg_sp.at[pl.ds(s * ZCH, ZCH)])
    plsc.subcore_barrier()

    base = E + wid * EPT

    def iload(g, buf, b, sem):
        off = base + (g * U + b) * K
        pltpu.async_copy(ei_hbm.at[pl.ds(off, K)], buf.at[b], sem)

    def idrain(buf, sem):
        pltpu.make_async_copy(ei_hbm.at[pl.ds(0, K)], buf.at[0], sem).wait()

    def scat(buf, b, sem):
        pltpu.async_copy(ones_v, deg_sp.at[buf.at[b]], sem, add=True)

    def sdrain(buf, sem):
        pltpu.make_async_copy(ones_v, deg_sp.at[buf.at[0]], sem).wait()

    for b in range(U):
        iload(0, idxA, b, semiA)

    def body(i, carry):
        g0 = 2 * i

        @pl.when(i > 0)
        def _():
            for b in range(U):
                sdrain(idxB, semsB)

        for b in range(U):
            iload(g0 + 1, idxB, b, semiB)
        for b in range(U):
            idrain(idxA, semiA)
        for b in range(U):
            scat(idxA, b, semsA)

        for b in range(U):
            idrain(idxB, semiB)
        for b in range(U):
            scat(idxB, b, semsB)
        for b in range(U):
            sdrain(idxA, semsA)
        for b in range(U):
            iload(g0 + 2, idxA, b, semiA)
        return carry

    lax.fori_loop(0, NPAIR, body, 0)

    for b in range(U):
        idrain(idxA, semiA)
    for b in range(U):
        scat(idxA, b, semsA)
    for b in range(U):
        sdrain(idxB, semsB)
    for b in range(U):
        sdrain(idxA, semsA)

    plsc.subcore_barrier()
    pltpu.sync_copy(deg_sp.at[pl.ds(s * ZCH, ZCH)], out_hbm.at[c, pl.ds(s * ZCH, ZCH)])


@functools.cache
def _deg_hist_kernel():
    return pl.kernel(
        _deg_hist_body,
        out_type=jax.ShapeDtypeStruct((NC, NPAD), jnp.float32),
        mesh=_sc_mesh(),
        scratch_types=[
            pltpu.VMEM((K,), jnp.float32),
            pltpu.VMEM((ZCH,), jnp.float32),
            pltpu.VMEM((U, K), jnp.int32),
            pltpu.VMEM((U, K), jnp.int32),
            pltpu.VMEM_SHARED((NPAD,), jnp.float32),
            pltpu.SemaphoreType.DMA,
            pltpu.SemaphoreType.DMA,
            pltpu.SemaphoreType.DMA,
            pltpu.SemaphoreType.DMA,
        ],
    )


def _deg_hist(ei):
    return _deg_hist_kernel()(ei)


def _edge_agg_body(ei_hbm, xs_hbm, out_hbm, *refs):
    idx = refs[0:ISLOT]
    rows = refs[ISLOT:ISLOT + RSLOT]
    agg_sp = refs[ISLOT + RSLOT]
    semi = refs[ISLOT + RSLOT + 1:ISLOT + RSLOT + 1 + ISLOT]
    semg = refs[ISLOT + RSLOT + 1 + ISLOT:ISLOT + RSLOT + 1 + ISLOT + RSLOT]
    sems = refs[ISLOT + RSLOT + 1 + ISLOT + RSLOT:]

    c = lax.axis_index("c")
    s = lax.axis_index("s")
    wid = c * NS + s
    base = wid * EPT

    def zrow(i, carry):
        for j in range(D_IN // 16):
            rows[0][i, pl.ds(j * 16, 16)] = jnp.zeros((16,), jnp.float32)
        return carry

    lax.fori_loop(0, K, zrow, 0)
    for z in range(ZCH // K):
        pltpu.sync_copy(rows[0], agg_sp.at[pl.ds(s * ZCH + z * K, K)])
    plsc.subcore_barrier()

    def eload(g, sj):
        sl = idx[sj % ISLOT]
        sem = semi[sj % ISLOT]
        off = base + g * K
        pltpu.async_copy(ei_hbm.at[pl.ds(off, K)], sl.at[0], sem)
        pltpu.async_copy(ei_hbm.at[pl.ds(E + off, K)], sl.at[1], sem)

    def edrain(sj):
        sl = idx[sj % ISLOT]
        sem = semi[sj % ISLOT]
        pltpu.make_async_copy(ei_hbm.at[pl.ds(0, K)], sl.at[0], sem).wait()
        pltpu.make_async_copy(ei_hbm.at[pl.ds(0, K)], sl.at[1], sem).wait()

    def gath(sj):
        pltpu.async_copy(xs_hbm.at[idx[sj % ISLOT].at[0]], rows[sj % RSLOT],
                         semg[sj % RSLOT])

    def gdrain(sj):
        pltpu.make_async_copy(xs_hbm.at[idx[sj % ISLOT].at[0]],
                              rows[sj % RSLOT], semg[sj % RSLOT]).wait()

    def scat(sj):
        pltpu.async_copy(rows[sj % RSLOT], agg_sp.at[idx[sj % ISLOT].at[1]],
                         sems[sj % RSLOT], add=True)

    def sdrain(sj):
        pltpu.make_async_copy(rows[sj % RSLOT],
                              agg_sp.at[idx[sj % ISLOT].at[1]],
                              sems[sj % RSLOT]).wait()

    for g in range(4):
        eload(g, g)
    edrain(0)
    edrain(1)
    gath(0)
    gath(1)

    def body(i, carry):
        for jj in range(ISLOT):
            g = i * ISLOT + jj
            if jj >= 2:
                sdrain(jj - 2)
            else:

                @pl.when(i > 0)
                def _():
                    sdrain(jj - 2)

            eload(g + 4, jj + 4)
            edrain(jj + 2)
            gath(jj + 2)
            gdrain(jj)
            scat(jj)
        return carry

    lax.fori_loop(0, NBODY, body, 0)

    for g in range(ETAIL, NCHUNK):
        sdrain(g - 2)
        if g + 4 < NCHUNK:
            eload(g + 4, g + 4)
        if g + 2 < NCHUNK:
            edrain(g + 2)
            gath(g + 2)
        gdrain(g)
        scat(g)
    sdrain(NCHUNK - 2)
    sdrain(NCHUNK - 1)

    plsc.subcore_barrier()
    pltpu.sync_copy(
        agg_sp.at[pl.ds(s * ZCH, ZCH)], out_hbm.at[c, pl.ds(s * ZCH, ZCH)]
    )


@functools.cache
def _edge_agg_kernel():
    scratch = (
        [pltpu.VMEM((2, K), jnp.int32) for _ in range(ISLOT)]
        + [pltpu.VMEM((K, D_IN), jnp.float32) for _ in range(RSLOT)]
        + [pltpu.VMEM_SHARED((NPAD, D_IN), jnp.float32)]
        + [pltpu.SemaphoreType.DMA for _ in range(ISLOT + 2 * RSLOT)]
    )
    return pl.kernel(
        _edge_agg_body,
        out_type=jax.ShapeDtypeStruct((NC, NPAD, D_IN), jnp.float32),
        mesh=_sc_mesh(),
        scratch_types=scratch,
    )


def _edge_agg(ei, xs):
    return _edge_agg_kernel()(ei, xs)


_RB = 2000
_GB = N // _RB


def _prescale_body(degT_ref, x_ref, xs_ref, dinv_ref):
    deg = degT_ref[:, 0:1] + degT_ref[:, 1:2] + 1.0
    dinv = lax.rsqrt(deg)
    dinv_ref[...] = dinv
    xs_ref[...] = x_ref[...] * dinv


_RP = 5000

def _prescale(degT, x):
    return pl.pallas_call(
        _prescale_body,
        grid=(N // _RP,),
        in_specs=[
            pl.BlockSpec((_RP, NC), lambda i: (i, 0)),
            pl.BlockSpec((_RP, D_IN), lambda i: (i, 0)),
        ],
        out_specs=[
            pl.BlockSpec((_RP, D_IN), lambda i: (i, 0)),
            pl.BlockSpec((_RP, 1), lambda i: (i, 0)),
        ],
        out_shape=[
            jax.ShapeDtypeStruct((N, D_IN), jnp.float32),
            jax.ShapeDtypeStruct((N, 1), jnp.float32),
        ],
    )(degT, x)


def _dense_body(agg_ref, xs_ref, dinv_ref, w_ref, bg_ref, g_ref, be_ref,
                wl_ref, bl_ref, probs_ref, rsu_ref, ca_ref, gram_ref,
                w2_ref, c2_ref, w3_ref, c3_ref, acache_ref):
    j = pl.program_id(0)

    @pl.when(j == 0)
    def _():
        ca_ref[...] = jnp.zeros_like(ca_ref)
        gram_ref[...] = jnp.zeros_like(gram_ref)

    @pl.when(j < _GB)
    def _():
        a = (agg_ref[0] + agg_ref[1] + xs_ref[...]) * dinv_ref[...]
        acache_ref[pl.ds(j * _RB, _RB), :] = a
        ca_ref[...] += jnp.sum(a, axis=0, keepdims=True)
        gram_ref[...] += lax.dot_general(
            a, a, (((0,), (0,)), ((), ())),
            preferred_element_type=jnp.float32)

    @pl.when(j == _GB)
    def _():
        inv_n = jnp.float32(1.0 / N)
        bg = bg_ref[...]
        m1 = jnp.dot(ca_ref[...] * inv_n, w_ref[...],
                     preferred_element_type=jnp.float32) + bg
        gw = jnp.dot(gram_ref[...], w_ref[...],
                     preferred_element_type=jnp.float32)
        q = jnp.sum(w_ref[...] * gw, axis=0, keepdims=True) * inv_n
        var = q + (2.0 * m1 - bg) * bg - m1 * m1
        scale = g_ref[...] * lax.rsqrt(var + EPS)
        w2 = w_ref[...] * scale
        c2 = (bg - m1) * scale + be_ref[...]
        w2_ref[...] = w2
        c2_ref[...] = c2
        w3_ref[...] = jnp.dot(w2, wl_ref[...],
                              preferred_element_type=jnp.float32)
        c3_ref[...] = jnp.dot(c2, wl_ref[...],
                              preferred_element_type=jnp.float32) + bl_ref[...]
        rsu_ref[...] = jnp.dot(acache_ref[0:1, :], w2,
                               preferred_element_type=jnp.float32) + c2

    @pl.when(j >= _GB)
    def _():
        a = acache_ref[pl.ds((j - _GB) * _RB, _RB), :]
        logits = jnp.dot(a, w3_ref[...], preferred_element_type=jnp.float32)
        logits = jnp.maximum(logits + c3_ref[...], 0.0)
        m = jnp.max(logits, axis=1, keepdims=True)
        e = jnp.exp(logits - m)
        probs_ref[...] = e / jnp.sum(e, axis=1, keepdims=True)


def _dense(agg, xs, dinv, W, bg, gamma, beta, Wl, bl):
    def row(j):
        return jnp.minimum(j, _GB - 1)

    return pl.pallas_call(
        _dense_body,
        grid=(2 * _GB,),
        in_specs=[
            pl.BlockSpec((NC, _RB, D_IN), lambda j: (0, row(j), 0)),
            pl.BlockSpec((_RB, D_IN), lambda j: (row(j), 0)),
            pl.BlockSpec((_RB, 1), lambda j: (row(j), 0)),
            pl.BlockSpec((D_IN, D_HID), lambda j: (0, 0)),
            pl.BlockSpec((1, D_HID), lambda j: (0, 0)),
            pl.BlockSpec((1, D_HID), lambda j: (0, 0)),
            pl.BlockSpec((1, D_HID), lambda j: (0, 0)),
            pl.BlockSpec((D_HID, D_OUT), lambda j: (0, 0)),
            pl.BlockSpec((1, D_OUT), lambda j: (0, 0)),
        ],
        out_specs=[
            pl.BlockSpec((_RB, D_OUT), lambda j: (jnp.where(j < _GB, 0, j - _GB), 0)),
            pl.BlockSpec((1, D_HID), lambda j: (0, 0)),
        ],
        out_shape=[
            jax.ShapeDtypeStruct((N, D_OUT), jnp.float32),
            jax.ShapeDtypeStruct((1, D_HID), jnp.float32),
        ],
        scratch_shapes=[
            pltpu.VMEM((1, D_IN), jnp.float32),
            pltpu.VMEM((D_IN, D_IN), jnp.float32),
            pltpu.VMEM((D_IN, D_HID), jnp.float32),
            pltpu.VMEM((1, D_HID), jnp.float32),
            pltpu.VMEM((D_IN, D_OUT), jnp.float32),
            pltpu.VMEM((1, D_OUT), jnp.float32),
            pltpu.VMEM((N, D_IN), jnp.float32),
        ],
    )(agg, xs, dinv, W, bg, gamma, beta, Wl, bl)


def kernel(node_feature, edge_index, W_gcn, b_gcn, gamma, beta, W_lin, b_lin):
    ei = edge_index.reshape(2 * E)
    deg_p = _deg_hist(ei)
    degT = deg_p.T
    xs, dinv = _prescale(degT, node_feature)
    agg = _edge_agg(ei, xs)
    probs, rsu = _dense(agg, xs, dinv, W_gcn, b_gcn.reshape(1, D_HID),
                        gamma.reshape(1, D_HID), beta.reshape(1, D_HID),
                        W_lin, b_lin.reshape(1, D_OUT))
    return probs, rsu
